# baseline (speedup 1.0000x reference)
import jax
import jax.numpy as jnp
from jax import lax
from jax.experimental import pallas as pl
from jax.experimental.pallas import tpu as pltpu
from jax.experimental.pallas import tpu_sc as plsc

_N = 10000
_E = 320000
_D = 128
_NC = 2
_NS = 16
_NW = _NC * _NS
_N_PAD = 10240
_RPT = _N_PAD // _NS
_K = 100
_EPT = _E // _NW
_NCH = _EPT // _K
_NBUF = 2
_NCHP = _NCH // 2
_ZB = 64


def _deg_body(dstm, zeros1, ones1, pdeg, sdeg, idxd, ones_v, *dsem):
    c = lax.axis_index("c")
    s = lax.axis_index("s")
    w = c * _NS + s
    pltpu.sync_copy(zeros1.at[pl.ds(s * _RPT, _RPT)], sdeg.at[pl.ds(s * _RPT, _RPT)])
    pltpu.sync_copy(ones1, ones_v)
    pltpu.sync_copy(dstm.at[pl.ds(w * _NCH, _NCH)], idxd)
    plsc.subcore_barrier()

    def sdesc(k, b):
        return pltpu.make_async_copy(ones_v, sdeg.at[idxd.at[k, 0]], dsem[b])

    def rnd(r, carry):
        for b in range(_NBUF):
            k = _NBUF * r + b

            @pl.when(k >= _NBUF)
            def _():
                sdesc(k - _NBUF, b).wait()

            sdesc(k, b).start(add=True)
        return carry

    lax.fori_loop(0, _NCH // _NBUF, rnd, 0)
    for b in range(_NBUF):
        sdesc(_NCH - _NBUF + b, b).wait()
    plsc.subcore_barrier()
    pltpu.sync_copy(sdeg.at[pl.ds(s * _RPT, _RPT)],
                    pdeg.at[pl.ds(c * _N_PAD + s * _RPT, _RPT)])


def _scat_body(g, srcm, dstm, zeros2, pout, acc, *sems):
    def scoped(idxs, idxd, rows):
        _scat_inner(g, srcm, dstm, zeros2, pout, acc, idxs, idxd, rows, sems)

    pl.run_scoped(
        scoped,
        idxs=pltpu.VMEM((_NCHP, 1, _K), jnp.int32),
        idxd=pltpu.VMEM((_NCHP, 1, _K), jnp.int32),
        rows=[pltpu.VMEM((_K, 1, _D), jnp.float32) for _ in range(_NBUF)],
    )


def _scat_inner(g, srcm, dstm, zeros2, pout, acc, idxs, idxd, rows, sems):
    gsem = sems[:_NBUF]
    ssem = sems[_NBUF:]
    c = lax.axis_index("c")
    s = lax.axis_index("s")
    w = c * _NS + s

    def zrow(j, carry):
        pltpu.sync_copy(zeros2, acc.at[pl.ds(s * _RPT + j * _ZB, _ZB)])
        return carry

    lax.fori_loop(0, _RPT // _ZB, zrow, 0)
    plsc.subcore_barrier()

    def gdesc(k, b):
        return pltpu.make_async_copy(g.at[idxs.at[k, 0]], rows[b], gsem[b])

    def sdesc(k, b):
        return pltpu.make_async_copy(rows[b], acc.at[idxd.at[k, 0]], ssem[b])

    for p in range(_NCH // _NCHP):
        pltpu.sync_copy(srcm.at[pl.ds(w * _NCH + p * _NCHP, _NCHP)], idxs)
        pltpu.sync_copy(dstm.at[pl.ds(w * _NCH + p * _NCHP, _NCHP)], idxd)
        gdesc(0, 0).start()
        gdesc(1, 1).start()

        def rnd(r, carry):
            for b in range(_NBUF):
                k = _NBUF * r + b
                bb = (b + 2) % _NBUF
                gdesc(k, b).wait()
                sdesc(k, b).start(add=True)

                @pl.when(k >= _NBUF - 2)
                def _():
                    sdesc(k + 2 - _NBUF, bb).wait()

                @pl.when(k + 2 < _NCHP)
                def _():
                    gdesc(k + 2, bb).start()
            return carry

        lax.fori_loop(0, _NCHP // _NBUF, rnd, 0)
        for j in range(_NCHP + 2 - _NBUF, _NCHP):
            sdesc(j, j % _NBUF).wait()
    plsc.subcore_barrier()
    def wrow(j, carry):
        pltpu.sync_copy(acc.at[pl.ds(s * _RPT + j * _ZB, _ZB)],
                        pout.at[pl.ds(c * _N_PAD + s * _RPT + j * _ZB, _ZB)])
        return carry

    lax.fori_loop(0, _RPT // _ZB, wrow, 0)


def _mm_body(x_ref, w_ref, h_ref):
    h_ref[...] = jnp.dot(x_ref[...], w_ref[...],
                         preferred_element_type=jnp.float32)


def _scale_body(h_ref, d2_ref, g_ref):
    d2 = d2_ref[...]
    dinv = lax.rsqrt(d2[:, 0] + d2[:, 1] + 1.0)
    g_ref[...] = (h_ref[...] * dinv[:, None])[:, None, :]


def _fin_body(pout_ref, g_ref, d2_ref, b_ref, gam_ref, bet_ref, o_ref):
    pc = pout_ref[...]
    ssum = pc[:_N_PAD] + pc[_N_PAD:] + g_ref[...]
    d2 = d2_ref[...]
    dinv = lax.rsqrt(d2[:, 0] + d2[:, 1] + 1.0)
    pre = ssum * dinv[:, None] + b_ref[...]
    rid = lax.broadcasted_iota(jnp.int32, (_N_PAD, _D), 0)
    m = rid < _N
    mean = jnp.sum(jnp.where(m, pre, 0.0), axis=0) / _N
    dev = jnp.where(m, pre - mean[None, :], 0.0)
    var = jnp.sum(dev * dev, axis=0) / _N
    o = (pre - mean[None, :]) * lax.rsqrt(var + 1e-5) * gam_ref[...] + bet_ref[...]
    o_ref[...] = jnp.maximum(o, 0.0)


def kernel(x, edge_index, W, b, gamma, beta):
    f32 = jnp.float32
    srcm = edge_index[0].reshape(_NW * _NCH, 1, _K)
    dstm = edge_index[1].reshape(_NW * _NCH, 1, _K)
    zeros1 = jnp.zeros((_N_PAD,), f32)
    ones1 = jnp.ones((_K,), f32)
    zeros2 = jnp.zeros((_ZB, 1, _D), f32)

    mesh = plsc.VectorSubcoreMesh(core_axis_name="c", subcore_axis_name="s",
                                  num_cores=_NC, num_subcores=_NS)
    dma = pltpu.SemaphoreType.DMA

    pdeg = pl.kernel(
        _deg_body,
        out_type=jax.ShapeDtypeStruct((2 * _N_PAD,), f32),
        mesh=mesh,
        scratch_types=[
            pltpu.VMEM_SHARED((_N_PAD,), f32),
            pltpu.VMEM((_NCH, 1, _K), jnp.int32),
            pltpu.VMEM((_K,), f32),
        ] + [dma] * _NBUF,
    )(dstm, zeros1, ones1)
    d2 = pdeg.reshape(2, _N_PAD).T

    x_pad = jnp.pad(x, ((0, _N_PAD - _N), (0, 0)))
    bn = 2048
    h = pl.pallas_call(
        _mm_body,
        grid=(_N_PAD // bn,),
        in_specs=[
            pl.BlockSpec((bn, _D), lambda i: (i, 0)),
            pl.BlockSpec((_D, _D), lambda i: (0, 0)),
        ],
        out_specs=pl.BlockSpec((bn, _D), lambda i: (i, 0)),
        out_shape=jax.ShapeDtypeStruct((_N_PAD, _D), f32),
    )(x_pad, W)
    g = pl.pallas_call(
        _scale_body,
        grid=(_N_PAD // bn,),
        in_specs=[
            pl.BlockSpec((bn, _D), lambda i: (i, 0)),
            pl.BlockSpec((bn, 2), lambda i: (i, 0)),
        ],
        out_specs=pl.BlockSpec((bn, 1, _D), lambda i: (i, 0, 0)),
        out_shape=jax.ShapeDtypeStruct((_N_PAD, 1, _D), f32),
    )(h, d2)

    pout = pl.kernel(
        _scat_body,
        out_type=jax.ShapeDtypeStruct((2 * _N_PAD, 1, _D), f32),
        mesh=mesh,
        scratch_types=[
            pltpu.VMEM_SHARED((_N_PAD, 1, _D), f32),
        ] + [dma] * (2 * _NBUF),
    )(g, srcm, dstm, zeros2)

    out = pl.pallas_call(
        _fin_body,
        out_shape=jax.ShapeDtypeStruct((_N_PAD, _D), f32),
    )(pout.reshape(2 * _N_PAD, _D), g.reshape(_N_PAD, _D), d2,
      b.reshape(1, _D), gamma.reshape(1, _D), beta.reshape(1, _D))
    return out[:_N]


# speedup vs baseline: 1.0271x; 1.0271x over previous
import jax
import jax.numpy as jnp
from jax import lax
from jax.experimental import pallas as pl
from jax.experimental.pallas import tpu as pltpu
from jax.experimental.pallas import tpu_sc as plsc

_N = 10000
_E = 320000
_D = 128
_NC = 2
_NS = 16
_NW = _NC * _NS
_N_PAD = 10240
_RPT = _N_PAD // _NS
_K = 100
_EPT = _E // _NW
_NCH = _EPT // _K
_NBUF = 2
_NCHP = _NCH // 2
_ZB = 64


def _deg_body(dstm, zeros1, ones1, pdeg, sdeg, idxd, ones_v, *dsem):
    c = lax.axis_index("c")
    s = lax.axis_index("s")
    w = c * _NS + s
    pltpu.sync_copy(zeros1.at[pl.ds(s * _RPT, _RPT)], sdeg.at[pl.ds(s * _RPT, _RPT)])
    pltpu.sync_copy(ones1, ones_v)
    pltpu.sync_copy(dstm.at[pl.ds(w * _NCH, _NCH)], idxd)
    plsc.subcore_barrier()

    def sdesc(k, b):
        return pltpu.make_async_copy(ones_v, sdeg.at[idxd.at[k, 0]], dsem[b])

    def rnd(r, carry):
        for b in range(_NBUF):
            k = _NBUF * r + b

            @pl.when(k >= _NBUF)
            def _():
                sdesc(k - _NBUF, b).wait()

            sdesc(k, b).start(add=True)
        return carry

    lax.fori_loop(0, _NCH // _NBUF, rnd, 0)
    for b in range(_NBUF):
        sdesc(_NCH - _NBUF + b, b).wait()
    plsc.subcore_barrier()
    pltpu.sync_copy(sdeg.at[pl.ds(s * _RPT, _RPT)],
                    pdeg.at[pl.ds(c * _N_PAD + s * _RPT, _RPT)])


def _scat_body(g, srcm, dstm, zeros2, pout, acc, *sems):
    def scoped(idxs, idxd, rows):
        _scat_inner(g, srcm, dstm, zeros2, pout, acc, idxs, idxd, rows, sems)

    pl.run_scoped(
        scoped,
        idxs=pltpu.VMEM((_NCHP, 1, _K), jnp.int32),
        idxd=pltpu.VMEM((_NCHP, 1, _K), jnp.int32),
        rows=[pltpu.VMEM((_K, 1, _D), jnp.float32) for _ in range(_NBUF)],
    )


def _scat_inner(g, srcm, dstm, zeros2, pout, acc, idxs, idxd, rows, sems):
    gsem = sems[:_NBUF]
    ssem = sems[_NBUF:]
    c = lax.axis_index("c")
    s = lax.axis_index("s")
    w = c * _NS + s

    def zrow(j, carry):
        pltpu.sync_copy(zeros2, acc.at[pl.ds(s * _RPT + j * _ZB, _ZB)])
        return carry

    lax.fori_loop(0, _RPT // _ZB, zrow, 0)
    plsc.subcore_barrier()

    def gdesc(k, b):
        return pltpu.make_async_copy(g.at[idxs.at[k, 0]], rows[b], gsem[b])

    def sdesc(k, b):
        return pltpu.make_async_copy(rows[b], acc.at[idxd.at[k, 0]], ssem[b])

    for p in range(_NCH // _NCHP):
        pltpu.sync_copy(srcm.at[pl.ds(w * _NCH + p * _NCHP, _NCHP)], idxs)
        pltpu.sync_copy(dstm.at[pl.ds(w * _NCH + p * _NCHP, _NCHP)], idxd)
        gdesc(0, 0).start()
        gdesc(1, 1).start()

        def rnd(r, carry):
            for b in range(_NBUF):
                k = _NBUF * r + b
                bb = (b + 2) % _NBUF
                gdesc(k, b).wait()
                sdesc(k, b).start(add=True)

                @pl.when(k >= _NBUF - 2)
                def _():
                    sdesc(k + 2 - _NBUF, bb).wait()

                @pl.when(k + 2 < _NCHP)
                def _():
                    gdesc(k + 2, bb).start()
            return carry

        lax.fori_loop(0, _NCHP // _NBUF, rnd, 0)
        for j in range(_NCHP + 2 - _NBUF, _NCHP):
            sdesc(j, j % _NBUF).wait()
    plsc.subcore_barrier()
    def wrow(j, carry):
        pltpu.sync_copy(acc.at[pl.ds(s * _RPT + j * _ZB, _ZB)],
                        pout.at[pl.ds(c * _N_PAD + s * _RPT + j * _ZB, _ZB)])
        return carry

    lax.fori_loop(0, _RPT // _ZB, wrow, 0)


def _mm_body(x_ref, w_ref, h_ref):
    h_ref[...] = jnp.dot(x_ref[...], w_ref[...],
                         preferred_element_type=jnp.float32)


def _scale_body(h_ref, pd_ref, g_ref):
    pd = pd_ref[...]
    dinv = lax.rsqrt(pd[0] + pd[1] + 1.0)
    g_ref[...] = (h_ref[...] * dinv[:, None])[:, None, :]


def _fin_body(pout_ref, g_ref, pd_ref, b_ref, gam_ref, bet_ref, o_ref):
    pc = pout_ref[...]
    ssum = pc[:_N] + pc[_N_PAD:_N_PAD + _N] + g_ref[...]
    pd = pd_ref[...]
    dinv = lax.rsqrt(pd[0, :_N] + pd[1, :_N] + 1.0)
    pre = ssum * dinv[:, None] + b_ref[...]
    mean = jnp.sum(pre, axis=0) / _N
    dev = pre - mean[None, :]
    var = jnp.sum(dev * dev, axis=0) / _N
    o = dev * lax.rsqrt(var + 1e-5) * gam_ref[...] + bet_ref[...]
    o_ref[...] = jnp.maximum(o, 0.0)


def kernel(x, edge_index, W, b, gamma, beta):
    f32 = jnp.float32
    srcm = edge_index[0].reshape(_NW * _NCH, 1, _K)
    dstm = edge_index[1].reshape(_NW * _NCH, 1, _K)
    zeros1 = jnp.zeros((_N_PAD,), f32)
    ones1 = jnp.ones((_K,), f32)
    zeros2 = jnp.zeros((_ZB, 1, _D), f32)

    mesh = plsc.VectorSubcoreMesh(core_axis_name="c", subcore_axis_name="s",
                                  num_cores=_NC, num_subcores=_NS)
    dma = pltpu.SemaphoreType.DMA

    pdeg = pl.kernel(
        _deg_body,
        out_type=jax.ShapeDtypeStruct((2 * _N_PAD,), f32),
        mesh=mesh,
        scratch_types=[
            pltpu.VMEM_SHARED((_N_PAD,), f32),
            pltpu.VMEM((_NCH, 1, _K), jnp.int32),
            pltpu.VMEM((_K,), f32),
        ] + [dma] * _NBUF,
    )(dstm, zeros1, ones1)
    pdeg2 = pdeg.reshape(2, _N_PAD)

    bn = 2048
    h = pl.pallas_call(
        _mm_body,
        grid=(_N_PAD // bn,),
        in_specs=[
            pl.BlockSpec((bn, _D), lambda i: (i, 0)),
            pl.BlockSpec((_D, _D), lambda i: (0, 0)),
        ],
        out_specs=pl.BlockSpec((bn, _D), lambda i: (i, 0)),
        out_shape=jax.ShapeDtypeStruct((_N, _D), f32),
    )(x, W)
    g = pl.pallas_call(
        _scale_body,
        grid=(_N_PAD // bn,),
        in_specs=[
            pl.BlockSpec((bn, _D), lambda i: (i, 0)),
            pl.BlockSpec((2, bn), lambda i: (0, i)),
        ],
        out_specs=pl.BlockSpec((bn, 1, _D), lambda i: (i, 0, 0)),
        out_shape=jax.ShapeDtypeStruct((_N, 1, _D), f32),
    )(h, pdeg2)

    pout = pl.kernel(
        _scat_body,
        out_type=jax.ShapeDtypeStruct((2 * _N_PAD, 1, _D), f32),
        mesh=mesh,
        scratch_types=[
            pltpu.VMEM_SHARED((_N_PAD, 1, _D), f32),
        ] + [dma] * (2 * _NBUF),
    )(g, srcm, dstm, zeros2)

    out = pl.pallas_call(
        _fin_body,
        out_shape=jax.ShapeDtypeStruct((_N, _D), f32),
    )(pout.reshape(2 * _N_PAD, _D), g.reshape(_N, _D), pdeg2,
      b.reshape(1, _D), gamma.reshape(1, _D), beta.reshape(1, _D))
    return out


# speedup vs baseline: 1.1010x; 1.0720x over previous
import jax
import jax.numpy as jnp
from jax import lax
from jax.experimental import pallas as pl
from jax.experimental.pallas import tpu as pltpu
from jax.experimental.pallas import tpu_sc as plsc

_N = 10000
_E = 320000
_D = 128
_NC = 2
_NS = 16
_NW = _NC * _NS
_N_PAD = 10240
_RPT = _N_PAD // _NS
_K = 128
_NROW = _E // _K
_NCH = _NROW // _NW
_NXW = _NROW - _NCH * _NW
_NBUF = 2
_NCHP = 40
_ZB = 64


def _deg_body(dstm, zeros1, ones1, pdeg, sdeg, idxd, exd, ones_v, *dsem):
    c = lax.axis_index("c")
    s = lax.axis_index("s")
    w = c * _NS + s
    pltpu.sync_copy(zeros1.at[pl.ds(s * _RPT, _RPT)], sdeg.at[pl.ds(s * _RPT, _RPT)])
    pltpu.sync_copy(ones1, ones_v)
    wbase = w * _NCH + jnp.minimum(w, _NXW)
    pltpu.sync_copy(dstm.at[pl.ds(wbase, _NCH)], idxd)

    @pl.when(w < _NXW)
    def _():
        pltpu.sync_copy(dstm.at[pl.ds(wbase + _NCH, 1)], exd)

    plsc.subcore_barrier()

    def sdesc(k, b):
        return pltpu.make_async_copy(ones_v, sdeg.at[idxd.at[k, 0]], dsem[b])

    def rnd(r, carry):
        for b in range(_NBUF):
            k = _NBUF * r + b

            @pl.when(k >= _NBUF)
            def _():
                sdesc(k - _NBUF, b).wait()

            sdesc(k, b).start(add=True)
        return carry

    lax.fori_loop(0, _NCH // _NBUF, rnd, 0)
    for b in range(_NBUF):
        sdesc(_NCH - _NBUF + b, b).wait()

    @pl.when(w < _NXW)
    def _():
        pltpu.sync_copy(ones_v, sdeg.at[exd.at[0, 0]], add=True)

    plsc.subcore_barrier()
    pltpu.sync_copy(sdeg.at[pl.ds(s * _RPT, _RPT)],
                    pdeg.at[pl.ds(c * _N_PAD + s * _RPT, _RPT)])


def _scat_body(g, srcm, dstm, zeros2, pout, acc, *sems):
    def scoped(idxs, idxd, exs, exd, rows):
        _scat_inner(g, srcm, dstm, zeros2, pout, acc, idxs, idxd, exs, exd,
                    rows, sems)

    pl.run_scoped(
        scoped,
        idxs=pltpu.VMEM((_NCHP, 1, _K), jnp.int32),
        idxd=pltpu.VMEM((_NCHP, 1, _K), jnp.int32),
        exs=pltpu.VMEM((1, 1, _K), jnp.int32),
        exd=pltpu.VMEM((1, 1, _K), jnp.int32),
        rows=[pltpu.VMEM((_K, 1, _D), jnp.float32) for _ in range(_NBUF)],
    )


def _scat_inner(g, srcm, dstm, zeros2, pout, acc, idxs, idxd, exs, exd, rows, sems):
    gsem = sems[:_NBUF]
    ssem = sems[_NBUF:]
    c = lax.axis_index("c")
    s = lax.axis_index("s")
    w = c * _NS + s

    def zrow(j, carry):
        pltpu.sync_copy(zeros2, acc.at[pl.ds(s * _RPT + j * _ZB, _ZB)])
        return carry

    lax.fori_loop(0, _RPT // _ZB, zrow, 0)
    plsc.subcore_barrier()

    def gdesc(k, b):
        return pltpu.make_async_copy(g.at[idxs.at[k, 0]], rows[b], gsem[b])

    def sdesc(k, b):
        return pltpu.make_async_copy(rows[b], acc.at[idxd.at[k, 0]], ssem[b])

    wbase = w * _NCH + jnp.minimum(w, _NXW)

    @pl.when(w < _NXW)
    def _():
        pltpu.sync_copy(srcm.at[pl.ds(wbase + _NCH, 1)], exs)
        pltpu.sync_copy(dstm.at[pl.ds(wbase + _NCH, 1)], exd)

    for off, n in ((0, _NCHP), (_NCHP, _NCH - _NCHP)):
        pltpu.sync_copy(srcm.at[pl.ds(wbase + off, n)], idxs.at[pl.ds(0, n)])
        pltpu.sync_copy(dstm.at[pl.ds(wbase + off, n)], idxd.at[pl.ds(0, n)])
        gdesc(0, 0).start()
        gdesc(1, 1).start()

        def rnd(r, carry):
            for b in range(_NBUF):
                k = _NBUF * r + b
                bb = (b + 2) % _NBUF
                gdesc(k, b).wait()
                sdesc(k, b).start(add=True)

                @pl.when(k >= _NBUF - 2)
                def _():
                    sdesc(k + 2 - _NBUF, bb).wait()

                @pl.when(k + 2 < n)
                def _():
                    gdesc(k + 2, bb).start()
            return carry

        lax.fori_loop(0, n // _NBUF, rnd, 0)
        for j in range(n + 2 - _NBUF, n):
            sdesc(j, j % _NBUF).wait()

    @pl.when(w < _NXW)
    def _():
        pltpu.sync_copy(g.at[exs.at[0, 0]], rows[0])
        pltpu.sync_copy(rows[0], acc.at[exd.at[0, 0]], add=True)

    plsc.subcore_barrier()
    def wrow(j, carry):
        pltpu.sync_copy(acc.at[pl.ds(s * _RPT + j * _ZB, _ZB)],
                        pout.at[pl.ds(c * _N_PAD + s * _RPT + j * _ZB, _ZB)])
        return carry

    lax.fori_loop(0, _RPT // _ZB, wrow, 0)


def _mm_body(x_ref, w_ref, h_ref):
    h_ref[...] = jnp.dot(x_ref[...], w_ref[...],
                         preferred_element_type=jnp.float32)


def _scale_body(h_ref, pd_ref, g_ref):
    pd = pd_ref[...]
    dinv = lax.rsqrt(pd[0] + pd[1] + 1.0)
    g_ref[...] = (h_ref[...] * dinv[:, None])[:, None, :]


def _fin_body(pout_ref, g_ref, pd_ref, b_ref, gam_ref, bet_ref, o_ref):
    pc = pout_ref[...]
    ssum = pc[:_N] + pc[_N_PAD:_N_PAD + _N] + g_ref[...]
    pd = pd_ref[...]
    dinv = lax.rsqrt(pd[0, :_N] + pd[1, :_N] + 1.0)
    pre = ssum * dinv[:, None] + b_ref[...]
    mean = jnp.sum(pre, axis=0) / _N
    dev = pre - mean[None, :]
    var = jnp.sum(dev * dev, axis=0) / _N
    o = dev * lax.rsqrt(var + 1e-5) * gam_ref[...] + bet_ref[...]
    o_ref[...] = jnp.maximum(o, 0.0)


def kernel(x, edge_index, W, b, gamma, beta):
    f32 = jnp.float32
    srcm = edge_index[0].reshape(_NROW, 1, _K)
    dstm = edge_index[1].reshape(_NROW, 1, _K)
    zeros1 = jnp.zeros((_N_PAD,), f32)
    ones1 = jnp.ones((_K,), f32)
    zeros2 = jnp.zeros((_ZB, 1, _D), f32)

    mesh = plsc.VectorSubcoreMesh(core_axis_name="c", subcore_axis_name="s",
                                  num_cores=_NC, num_subcores=_NS)
    dma = pltpu.SemaphoreType.DMA

    pdeg = pl.kernel(
        _deg_body,
        out_type=jax.ShapeDtypeStruct((2 * _N_PAD,), f32),
        mesh=mesh,
        scratch_types=[
            pltpu.VMEM_SHARED((_N_PAD,), f32),
            pltpu.VMEM((_NCH, 1, _K), jnp.int32),
            pltpu.VMEM((1, 1, _K), jnp.int32),
            pltpu.VMEM((_K,), f32),
        ] + [dma] * _NBUF,
    )(dstm, zeros1, ones1)
    pdeg2 = pdeg.reshape(2, _N_PAD)

    bn = 2048
    h = pl.pallas_call(
        _mm_body,
        grid=(_N_PAD // bn,),
        in_specs=[
            pl.BlockSpec((bn, _D), lambda i: (i, 0)),
            pl.BlockSpec((_D, _D), lambda i: (0, 0)),
        ],
        out_specs=pl.BlockSpec((bn, _D), lambda i: (i, 0)),
        out_shape=jax.ShapeDtypeStruct((_N, _D), f32),
    )(x, W)
    g = pl.pallas_call(
        _scale_body,
        grid=(_N_PAD // bn,),
        in_specs=[
            pl.BlockSpec((bn, _D), lambda i: (i, 0)),
            pl.BlockSpec((2, bn), lambda i: (0, i)),
        ],
        out_specs=pl.BlockSpec((bn, 1, _D), lambda i: (i, 0, 0)),
        out_shape=jax.ShapeDtypeStruct((_N, 1, _D), f32),
    )(h, pdeg2)

    pout = pl.kernel(
        _scat_body,
        out_type=jax.ShapeDtypeStruct((2 * _N_PAD, 1, _D), f32),
        mesh=mesh,
        scratch_types=[
            pltpu.VMEM_SHARED((_N_PAD, 1, _D), f32),
        ] + [dma] * (2 * _NBUF),
    )(g, srcm, dstm, zeros2)

    out = pl.pallas_call(
        _fin_body,
        out_shape=jax.ShapeDtypeStruct((_N, _D), f32),
    )(pout.reshape(2 * _N_PAD, _D), g.reshape(_N, _D), pdeg2,
      b.reshape(1, _D), gamma.reshape(1, _D), beta.reshape(1, _D))
    return out


# speedup vs baseline: 1.1050x; 1.0036x over previous
import jax
import jax.numpy as jnp
from jax import lax
from jax.experimental import pallas as pl
from jax.experimental.pallas import tpu as pltpu
from jax.experimental.pallas import tpu_sc as plsc

_N = 10000
_E = 320000
_D = 128
_NC = 2
_NS = 16
_NW = _NC * _NS
_N_PAD = 10240
_RPT = _N_PAD // _NS
_K = 128
_NROW = _E // _K
_NCH = _NROW // _NW
_NXW = _NROW - _NCH * _NW
_NBUF = 2
_NCHP = 40
_ZB = 64


def _deg_body(dstm, zeros1, ones1, pdeg, sdeg, idxd, exd, ones_v, *dsem):
    c = lax.axis_index("c")
    s = lax.axis_index("s")
    w = c * _NS + s
    pltpu.sync_copy(zeros1.at[pl.ds(s * _RPT, _RPT)], sdeg.at[pl.ds(s * _RPT, _RPT)])
    pltpu.sync_copy(ones1, ones_v)
    wbase = w * _NCH + jnp.minimum(w, _NXW)
    pltpu.sync_copy(dstm.at[pl.ds(wbase, _NCH)], idxd)

    @pl.when(w < _NXW)
    def _():
        pltpu.sync_copy(dstm.at[pl.ds(wbase + _NCH, 1)], exd)

    plsc.subcore_barrier()

    def sdesc(k, b):
        return pltpu.make_async_copy(ones_v, sdeg.at[idxd.at[k, 0]], dsem[b])

    def rnd(r, carry):
        for b in range(_NBUF):
            k = _NBUF * r + b

            @pl.when(k >= _NBUF)
            def _():
                sdesc(k - _NBUF, b).wait()

            sdesc(k, b).start(add=True)
        return carry

    lax.fori_loop(0, _NCH // _NBUF, rnd, 0)
    for b in range(_NBUF):
        sdesc(_NCH - _NBUF + b, b).wait()

    @pl.when(w < _NXW)
    def _():
        pltpu.sync_copy(ones_v, sdeg.at[exd.at[0, 0]], add=True)

    plsc.subcore_barrier()
    pltpu.sync_copy(sdeg.at[pl.ds(s * _RPT, _RPT)],
                    pdeg.at[pl.ds(c * _N_PAD + s * _RPT, _RPT)])


def _scat_body(g, srcm, dstm, zeros2, pout, acc, *sems):
    def scoped(idxs, idxd, exs, exd, rows):
        _scat_inner(g, srcm, dstm, zeros2, pout, acc, idxs, idxd, exs, exd,
                    rows, sems)

    pl.run_scoped(
        scoped,
        idxs=pltpu.VMEM((_NCHP, 1, _K), jnp.int32),
        idxd=pltpu.VMEM((_NCHP, 1, _K), jnp.int32),
        exs=pltpu.VMEM((1, 1, _K), jnp.int32),
        exd=pltpu.VMEM((1, 1, _K), jnp.int32),
        rows=[pltpu.VMEM((_K, _D), jnp.float32) for _ in range(_NBUF)],
    )


def _scat_inner(g, srcm, dstm, zeros2, pout, acc, idxs, idxd, exs, exd, rows, sems):
    gsem = sems[:_NBUF]
    ssem = sems[_NBUF:]
    c = lax.axis_index("c")
    s = lax.axis_index("s")
    w = c * _NS + s

    def zrow(j, carry):
        pltpu.sync_copy(zeros2, acc.at[pl.ds(s * _RPT + j * _ZB, _ZB)])
        return carry

    lax.fori_loop(0, _RPT // _ZB, zrow, 0)
    plsc.subcore_barrier()

    def gdesc(k, b):
        return pltpu.make_async_copy(g.at[idxs.at[k, 0]], rows[b], gsem[b])

    def sdesc(k, b):
        return pltpu.make_async_copy(rows[b], acc.at[idxd.at[k, 0]], ssem[b])

    wbase = w * _NCH + jnp.minimum(w, _NXW)

    @pl.when(w < _NXW)
    def _():
        pltpu.sync_copy(srcm.at[pl.ds(wbase + _NCH, 1)], exs)
        pltpu.sync_copy(dstm.at[pl.ds(wbase + _NCH, 1)], exd)

    for off, n in ((0, _NCHP), (_NCHP, _NCH - _NCHP)):
        pltpu.sync_copy(srcm.at[pl.ds(wbase + off, n)], idxs.at[pl.ds(0, n)])
        pltpu.sync_copy(dstm.at[pl.ds(wbase + off, n)], idxd.at[pl.ds(0, n)])
        gdesc(0, 0).start()
        gdesc(1, 1).start()

        def rnd(r, carry):
            for b in range(_NBUF):
                k = _NBUF * r + b
                bb = (b + 2) % _NBUF
                gdesc(k, b).wait()
                sdesc(k, b).start(add=True)

                @pl.when(k >= _NBUF - 2)
                def _():
                    sdesc(k + 2 - _NBUF, bb).wait()

                @pl.when(k + 2 < n)
                def _():
                    gdesc(k + 2, bb).start()
            return carry

        lax.fori_loop(0, n // _NBUF, rnd, 0)
        for j in range(n + 2 - _NBUF, n):
            sdesc(j, j % _NBUF).wait()

    @pl.when(w < _NXW)
    def _():
        pltpu.sync_copy(g.at[exs.at[0, 0]], rows[0])
        pltpu.sync_copy(rows[0], acc.at[exd.at[0, 0]], add=True)

    plsc.subcore_barrier()
    def wrow(j, carry):
        pltpu.sync_copy(acc.at[pl.ds(s * _RPT + j * _ZB, _ZB)],
                        pout.at[pl.ds(c * _N_PAD + s * _RPT + j * _ZB, _ZB)])
        return carry

    lax.fori_loop(0, _RPT // _ZB, wrow, 0)


def _mm_body(x_ref, w_ref, h_ref):
    h_ref[...] = jnp.dot(x_ref[...], w_ref[...],
                         preferred_element_type=jnp.float32)


def _scale_body(h_ref, pd_ref, g_ref):
    pd = pd_ref[...]
    dinv = lax.rsqrt(pd[0] + pd[1] + 1.0)
    g_ref[...] = h_ref[...] * dinv[:, None]


def _fin_body(pout_ref, g_ref, pd_ref, b_ref, gam_ref, bet_ref, o_ref):
    pc = pout_ref[...]
    ssum = pc[:_N] + pc[_N_PAD:_N_PAD + _N] + g_ref[...]
    pd = pd_ref[...]
    dinv = lax.rsqrt(pd[0, :_N] + pd[1, :_N] + 1.0)
    pre = ssum * dinv[:, None] + b_ref[...]
    mean = jnp.sum(pre, axis=0) / _N
    dev = pre - mean[None, :]
    var = jnp.sum(dev * dev, axis=0) / _N
    o = dev * lax.rsqrt(var + 1e-5) * gam_ref[...] + bet_ref[...]
    o_ref[...] = jnp.maximum(o, 0.0)


def kernel(x, edge_index, W, b, gamma, beta):
    f32 = jnp.float32
    srcm = edge_index[0].reshape(_NROW, 1, _K)
    dstm = edge_index[1].reshape(_NROW, 1, _K)
    zeros1 = jnp.zeros((_N_PAD,), f32)
    ones1 = jnp.ones((_K,), f32)
    zeros2 = jnp.zeros((_ZB, _D), f32)

    mesh = plsc.VectorSubcoreMesh(core_axis_name="c", subcore_axis_name="s",
                                  num_cores=_NC, num_subcores=_NS)
    dma = pltpu.SemaphoreType.DMA

    pdeg = pl.kernel(
        _deg_body,
        out_type=jax.ShapeDtypeStruct((2 * _N_PAD,), f32),
        mesh=mesh,
        scratch_types=[
            pltpu.VMEM_SHARED((_N_PAD,), f32),
            pltpu.VMEM((_NCH, 1, _K), jnp.int32),
            pltpu.VMEM((1, 1, _K), jnp.int32),
            pltpu.VMEM((_K,), f32),
        ] + [dma] * _NBUF,
    )(dstm, zeros1, ones1)
    pdeg2 = pdeg.reshape(2, _N_PAD)

    bn = 2048
    h = pl.pallas_call(
        _mm_body,
        grid=(_N_PAD // bn,),
        in_specs=[
            pl.BlockSpec((bn, _D), lambda i: (i, 0)),
            pl.BlockSpec((_D, _D), lambda i: (0, 0)),
        ],
        out_specs=pl.BlockSpec((bn, _D), lambda i: (i, 0)),
        out_shape=jax.ShapeDtypeStruct((_N, _D), f32),
    )(x, W)
    g = pl.pallas_call(
        _scale_body,
        grid=(_N_PAD // bn,),
        in_specs=[
            pl.BlockSpec((bn, _D), lambda i: (i, 0)),
            pl.BlockSpec((2, bn), lambda i: (0, i)),
        ],
        out_specs=pl.BlockSpec((bn, _D), lambda i: (i, 0)),
        out_shape=jax.ShapeDtypeStruct((_N, _D), f32),
    )(h, pdeg2)

    pout = pl.kernel(
        _scat_body,
        out_type=jax.ShapeDtypeStruct((2 * _N_PAD, _D), f32),
        mesh=mesh,
        scratch_types=[
            pltpu.VMEM_SHARED((_N_PAD, _D), f32),
        ] + [dma] * (2 * _NBUF),
    )(g, srcm, dstm, zeros2)

    out = pl.pallas_call(
        _fin_body,
        out_shape=jax.ShapeDtypeStruct((_N, _D), f32),
    )(pout, g, pdeg2,
      b.reshape(1, _D), gamma.reshape(1, _D), beta.reshape(1, _D))
    return out
